# baseline (speedup 1.0000x reference)
import functools

import jax
import jax.numpy as jnp
from jax import lax
from jax.experimental import pallas as pl
from jax.experimental.pallas import tpu as pltpu
from jax.experimental.pallas import tpu_sc as plsc

VOCAB = 1000000
EMBED_DIM = 64
BATCH = 16384
HIST = 200
PAD_DIM = 72

NUM_CORES = 2
NUM_SUBCORES = 16
NUM_WORKERS = NUM_CORES * NUM_SUBCORES

B = BATCH * HIST
B_PER_W = B // NUM_WORKERS
CHUNK = 400
N_CHUNKS = B_PER_W // CHUNK

assert B_PER_W * NUM_WORKERS == B
assert N_CHUNKS * CHUNK == B_PER_W
assert N_CHUNKS % 4 == 0 and N_CHUNKS >= 12
assert CHUNK % 8 == 0
assert 4 * CHUNK * EMBED_DIM * 4 + 4 * CHUNK * 4 < 524284


def _make_kernel():
    mesh = plsc.VectorSubcoreMesh(core_axis_name="c", subcore_axis_name="s")

    @functools.partial(
        pl.kernel,
        mesh=mesh,
        out_type=jax.ShapeDtypeStruct((B, PAD_DIM), jnp.float32),
        scratch_types=[
            pltpu.VMEM((CHUNK,), jnp.int32),
            pltpu.VMEM((CHUNK,), jnp.int32),
            pltpu.VMEM((CHUNK,), jnp.int32),
            pltpu.VMEM((CHUNK,), jnp.int32),
            pltpu.VMEM((CHUNK, EMBED_DIM), jnp.float32),
            pltpu.VMEM((CHUNK, EMBED_DIM), jnp.float32),
            pltpu.VMEM((CHUNK, EMBED_DIM), jnp.float32),
            pltpu.VMEM((CHUNK, EMBED_DIM), jnp.float32),
            pltpu.SemaphoreType.DMA,
            pltpu.SemaphoreType.DMA,
            pltpu.SemaphoreType.DMA,
            pltpu.SemaphoreType.DMA,
            pltpu.SemaphoreType.DMA,
            pltpu.SemaphoreType.DMA,
            pltpu.SemaphoreType.DMA,
            pltpu.SemaphoreType.DMA,
            pltpu.SemaphoreType.DMA,
            pltpu.SemaphoreType.DMA,
            pltpu.SemaphoreType.DMA,
            pltpu.SemaphoreType.DMA,
        ],
        compiler_params=pltpu.CompilerParams(use_tc_tiling_on_sc=False),
    )
    def gather_kernel(idx_hbm, table_hbm, out_hbm,
                      i0, i1, i2, i3, g0, g1, g2, g3,
                      si0, si1, si2, si3,
                      sg0, sg1, sg2, sg3,
                      sw0, sw1, sw2, sw3):
        ibuf = [i0, i1, i2, i3]
        gbuf = [g0, g1, g2, g3]
        sem_i = [si0, si1, si2, si3]
        sem_g = [sg0, sg1, sg2, sg3]
        sem_w = [sw0, sw1, sw2, sw3]

        wid = lax.axis_index("s") * NUM_CORES + lax.axis_index("c")
        base = wid * B_PER_W

        def idx_start(g, b):
            pltpu.async_copy(idx_hbm.at[pl.ds(base + g * CHUNK, CHUNK)],
                             ibuf[b], sem_i[b])

        def idx_wait(b):
            pltpu.make_async_copy(idx_hbm.at[pl.ds(base, CHUNK)],
                                  ibuf[b], sem_i[b]).wait()

        def gather_start(bi, bg):
            pltpu.async_copy(table_hbm.at[ibuf[bi]], gbuf[bg], sem_g[bg])

        def gather_wait(bi, bg):
            pltpu.make_async_copy(table_hbm.at[ibuf[bi]],
                                  gbuf[bg], sem_g[bg]).wait()

        def wb_start(g, b):
            pltpu.async_copy(
                gbuf[b],
                out_hbm.at[pl.ds(base + g * CHUNK, CHUNK),
                           pl.ds(0, EMBED_DIM)],
                sem_w[b])

        def wb_wait(b):
            pltpu.make_async_copy(
                gbuf[b],
                out_hbm.at[pl.ds(base, CHUNK), pl.ds(0, EMBED_DIM)],
                sem_w[b]).wait()

        for j in range(4):
            idx_start(j, j)
        idx_wait(0)
        gather_start(0, 0)
        idx_wait(1)
        gather_start(1, 1)
        for j in range(4):
            gather_wait(j % 4, j % 4)
            wb_start(j, j % 4)
            idx_wait((j + 2) % 4)
            if j >= 2:
                wb_wait((j + 2) % 4)
            gather_start((j + 2) % 4, (j + 2) % 4)
            idx_start(j + 4, j)

        def quad(k, carry):
            for j in range(4):
                g = 4 * k + j
                gather_wait(j % 4, j % 4)
                wb_start(g, j % 4)
                idx_wait((j + 2) % 4)
                wb_wait((j + 2) % 4)
                gather_start((j + 2) % 4, (j + 2) % 4)
                idx_start(jnp.minimum(g + 4, N_CHUNKS - 1), j)
            return carry

        lax.fori_loop(1, N_CHUNKS // 4, quad, 0)

        wb_wait(2)
        wb_wait(3)
        gather_wait(0, 0)
        gather_wait(1, 1)
        idx_wait(2)
        idx_wait(3)

    return gather_kernel


_gather = _make_kernel()


def kernel(product_ids, table):
    idx = product_ids.reshape(B).astype(jnp.int32)
    rows = _gather(idx, table)
    return rows[:, :EMBED_DIM].reshape(BATCH, HIST, EMBED_DIM)


# speedup vs baseline: 1.9223x; 1.9223x over previous
import functools

import jax
import jax.numpy as jnp
from jax import lax
from jax.experimental import pallas as pl
from jax.experimental.pallas import tpu as pltpu
from jax.experimental.pallas import tpu_sc as plsc

VOCAB = 1000000
EMBED_DIM = 64
BATCH = 16384
HIST = 200
PAD_DIM = 128

NUM_CORES = 2
NUM_SUBCORES = 16
NUM_WORKERS = NUM_CORES * NUM_SUBCORES

B = BATCH * HIST
B_PER_W = B // NUM_WORKERS
CHUNK = 400
N_CHUNKS = B_PER_W // CHUNK

assert B_PER_W * NUM_WORKERS == B
assert N_CHUNKS * CHUNK == B_PER_W
assert N_CHUNKS % 4 == 0 and N_CHUNKS >= 12
assert CHUNK % 8 == 0
assert 4 * CHUNK * EMBED_DIM * 4 + 4 * CHUNK * 4 < 524284


def _make_kernel():
    mesh = plsc.VectorSubcoreMesh(core_axis_name="c", subcore_axis_name="s")

    @functools.partial(
        pl.kernel,
        mesh=mesh,
        out_type=jax.ShapeDtypeStruct((B, PAD_DIM), jnp.float32),
        scratch_types=[
            pltpu.VMEM((CHUNK,), jnp.int32),
            pltpu.VMEM((CHUNK,), jnp.int32),
            pltpu.VMEM((CHUNK,), jnp.int32),
            pltpu.VMEM((CHUNK,), jnp.int32),
            pltpu.VMEM((CHUNK, EMBED_DIM), jnp.float32),
            pltpu.VMEM((CHUNK, EMBED_DIM), jnp.float32),
            pltpu.VMEM((CHUNK, EMBED_DIM), jnp.float32),
            pltpu.VMEM((CHUNK, EMBED_DIM), jnp.float32),
            pltpu.SemaphoreType.DMA,
            pltpu.SemaphoreType.DMA,
            pltpu.SemaphoreType.DMA,
            pltpu.SemaphoreType.DMA,
            pltpu.SemaphoreType.DMA,
            pltpu.SemaphoreType.DMA,
            pltpu.SemaphoreType.DMA,
            pltpu.SemaphoreType.DMA,
            pltpu.SemaphoreType.DMA,
            pltpu.SemaphoreType.DMA,
            pltpu.SemaphoreType.DMA,
            pltpu.SemaphoreType.DMA,
        ],
        compiler_params=pltpu.CompilerParams(use_tc_tiling_on_sc=False),
    )
    def gather_kernel(idx_hbm, table_hbm, out_hbm,
                      i0, i1, i2, i3, g0, g1, g2, g3,
                      si0, si1, si2, si3,
                      sg0, sg1, sg2, sg3,
                      sw0, sw1, sw2, sw3):
        ibuf = [i0, i1, i2, i3]
        gbuf = [g0, g1, g2, g3]
        sem_i = [si0, si1, si2, si3]
        sem_g = [sg0, sg1, sg2, sg3]
        sem_w = [sw0, sw1, sw2, sw3]

        wid = lax.axis_index("s") * NUM_CORES + lax.axis_index("c")
        base = wid * B_PER_W

        def idx_start(g, b):
            pltpu.async_copy(idx_hbm.at[pl.ds(base + g * CHUNK, CHUNK)],
                             ibuf[b], sem_i[b])

        def idx_wait(b):
            pltpu.make_async_copy(idx_hbm.at[pl.ds(base, CHUNK)],
                                  ibuf[b], sem_i[b]).wait()

        def gather_start(bi, bg):
            pltpu.async_copy(table_hbm.at[ibuf[bi]], gbuf[bg], sem_g[bg])

        def gather_wait(bi, bg):
            pltpu.make_async_copy(table_hbm.at[ibuf[bi]],
                                  gbuf[bg], sem_g[bg]).wait()

        def wb_start(g, b):
            pltpu.async_copy(
                gbuf[b],
                out_hbm.at[pl.ds(base + g * CHUNK, CHUNK),
                           pl.ds(0, EMBED_DIM)],
                sem_w[b])

        def wb_wait(b):
            pltpu.make_async_copy(
                gbuf[b],
                out_hbm.at[pl.ds(base, CHUNK), pl.ds(0, EMBED_DIM)],
                sem_w[b]).wait()

        for j in range(4):
            idx_start(j, j)
        idx_wait(0)
        gather_start(0, 0)
        idx_wait(1)
        gather_start(1, 1)
        for j in range(4):
            gather_wait(j % 4, j % 4)
            wb_start(j, j % 4)
            idx_wait((j + 2) % 4)
            if j >= 2:
                wb_wait((j + 2) % 4)
            gather_start((j + 2) % 4, (j + 2) % 4)
            idx_start(j + 4, j)

        def quad(k, carry):
            for j in range(4):
                g = 4 * k + j
                gather_wait(j % 4, j % 4)
                wb_start(g, j % 4)
                idx_wait((j + 2) % 4)
                wb_wait((j + 2) % 4)
                gather_start((j + 2) % 4, (j + 2) % 4)
                idx_start(jnp.minimum(g + 4, N_CHUNKS - 1), j)
            return carry

        lax.fori_loop(1, N_CHUNKS // 4, quad, 0)

        wb_wait(2)
        wb_wait(3)
        gather_wait(0, 0)
        gather_wait(1, 1)
        idx_wait(2)
        idx_wait(3)

    return gather_kernel


_gather = _make_kernel()


def kernel(product_ids, table):
    idx = product_ids.reshape(B).astype(jnp.int32)
    rows = _gather(idx, table)
    return rows[:, :EMBED_DIM].reshape(BATCH, HIST, EMBED_DIM)
